# baseline (speedup 1.0000x reference)
import functools

import jax
import jax.numpy as jnp
from jax import lax
from jax.experimental import pallas as pl
from jax.experimental.pallas import tpu as pltpu
from jax.experimental.pallas import tpu_sc as plsc

F32 = jnp.float32

_H = 128
_EH = 64
_NS = 16
_NC = 2
_NBKT = 4
_CHUNK = 64
_B = 12512
_ACC = 12544
_STRIPE = _ACC // _NS
_ZR = 16

_K1 = 0.7978845608028654
_K2 = _K1 * 0.044715


def _gelu_tc(v):
    return jax.nn.gelu(v)


def _enc_body(x_ref, wn_ref, bn_ref, o_ref):
    o_ref[...] = _gelu_tc(
        jnp.dot(x_ref[...], wn_ref[...], preferred_element_type=F32)
        + bn_ref[...])


def _ce_body(ea_ref, we_ref, be_ref, wme_ref, bm_ref, o_ref):
    L = wme_ref.shape[0]
    e = _gelu_tc(
        jnp.dot(ea_ref[...], we_ref[...], preferred_element_type=F32)
        + be_ref[...])
    for l in range(L):
        o_ref[l] = jnp.dot(e, wme_ref[l], preferred_element_type=F32) + bm_ref[l]


def _prep_body(h_ref, ws_ref, wd_ref, hs_ref, hd_ref):
    h = h_ref[...]
    hs_ref[...] = jnp.dot(h, ws_ref[...], preferred_element_type=F32)
    hd_ref[...] = jnp.dot(h, wd_ref[...], preferred_element_type=F32)


def _upd_body(h_ref, agg_ref, wu_ref, bu_ref, o_ref):
    h = h_ref[...]
    w = wu_ref[...]
    acc = (jnp.dot(h, w[:_H], preferred_element_type=F32)
           + jnp.dot(agg_ref[...], w[_H:], preferred_element_type=F32)
           + bu_ref[...])
    o_ref[...] = _gelu_tc(acc) + h


def _readout_body(h_ref, wp_ref, bp_ref, wv1_ref, bv1_ref, wv2_ref, bv2_ref,
                  lo_ref, vo_ref, sum_ref, max_ref, *, nblocks, n_rows):
    i = pl.program_id(0)
    blk = h_ref[...]
    bsum = jnp.sum(blk, axis=0, keepdims=True)
    bmax = jnp.max(blk, axis=0, keepdims=True)

    @pl.when(i == 0)
    def _():
        sum_ref[...] = bsum
        max_ref[...] = bmax

    @pl.when(i > 0)
    def _():
        sum_ref[...] = sum_ref[...] + bsum
        max_ref[...] = jnp.maximum(max_ref[...], bmax)

    @pl.when(i == nblocks - 1)
    def _():
        g = jnp.concatenate([sum_ref[...] / float(n_rows), max_ref[...]],
                            axis=1)
        lo_ref[...] = (jnp.dot(g, wp_ref[...], preferred_element_type=F32)
                       + bp_ref[...])
        v = _gelu_tc(jnp.dot(g, wv1_ref[...], preferred_element_type=F32)
                     + bv1_ref[...])
        vo_ref[...] = (jnp.dot(v, wv2_ref[...], preferred_element_type=F32)
                       + bv2_ref[...])


def _full_spec(shape):
    return pl.BlockSpec(shape, lambda i: tuple(0 for _ in shape))


def _make_edge_pass(N, E):
    assert E % _CHUNK == 0 and N <= _NBKT * _B
    nch = E // _CHUNK
    base_nj, rem = divmod(nch, _NS)
    mesh = plsc.VectorSubcoreMesh(core_axis_name="c", subcore_axis_name="s")

    @functools.partial(
        pl.kernel,
        out_type=jax.ShapeDtypeStruct((_NBKT * _B, _H), F32),
        mesh=mesh,
        scratch_types=[
            pltpu.VMEM((_CHUNK,), jnp.int32),
            pltpu.VMEM((1, _CHUNK), jnp.int32),
            pltpu.VMEM((1, _CHUNK), jnp.int32),
            pltpu.VMEM((_CHUNK, _H), F32),
            pltpu.VMEM((_CHUNK, _H), F32),
            pltpu.VMEM((_CHUNK, _H), F32),
            pltpu.VMEM((_ZR, _H), F32),
            pltpu.VMEM_SHARED((_ACC, _H), F32),
            pltpu.SemaphoreType.DMA,
        ],
    )
    def edge_pass(src_hbm, dst_hbm, hs_hbm, hd_hbm, ce_hbm, agg_hbm,
                  srcv, dstv, lidv, cev, hsv, hdv, zbuf, acc, sem):
        c = lax.axis_index("c")
        s = lax.axis_index("s")
        nj = base_nj + jnp.where(s < rem, 1, 0)

        zv = jnp.zeros((16,), F32)

        def zb(r, carry):
            for cb in range(_H // 16):
                zbuf[r, pl.ds(cb * 16, 16)] = zv
            return carry

        lax.fori_loop(0, _ZR, zb, 0)
        nzc = _STRIPE // _ZR

        for p in range(_NBKT // _NC):
            bkt = p * _NC + c
            bbase = bkt * _B

            def zc(t, carry):
                pltpu.sync_copy(zbuf, acc.at[pl.ds(s * _STRIPE + t * _ZR,
                                                   _ZR)])
                return carry

            lax.fori_loop(0, nzc, zc, 0)
            plsc.subcore_barrier()

            def chunk_body(j, carry):
                base = (s + _NS * j) * _CHUNK
                cp_s = pltpu.async_copy(src_hbm.at[pl.ds(base, _CHUNK)],
                                        srcv, sem)
                cp_d = pltpu.async_copy(dst_hbm.at[pl.ds(base, _CHUNK)],
                                        dstv.at[0], sem)
                cp_c = pltpu.async_copy(ce_hbm.at[pl.ds(base, _CHUNK)],
                                        cev, sem)
                cp_s.wait()
                cp_d.wait()

                def lids(t, carry2):
                    sl = pl.ds(t * 16, 16)
                    lid = dstv[0, sl] - bbase
                    ok = (lid >= 0) & (lid < _B)
                    lidv[0, sl] = jnp.where(ok, lid, _B)
                    return carry2

                lax.fori_loop(0, _CHUNK // 16, lids, 0)
                cp_hs = pltpu.async_copy(hs_hbm.at[srcv], hsv, sem)
                cp_hd = pltpu.async_copy(hd_hbm.at[dstv.at[0]], hdv, sem)
                cp_c.wait()
                cp_hs.wait()
                cp_hd.wait()

                def rows(r, carry2):
                    for cb in range(_H // 16):
                        sl = pl.ds(cb * 16, 16)
                        z = hsv[r, sl] + hdv[r, sl] + cev[r, sl]
                        az = jnp.abs(z)
                        t_ = _K1 + _K2 * (z * z)
                        eu = jnp.exp((az + az) * t_)
                        rr = 1.0 / (eu + 1.0)
                        hsv[r, sl] = jnp.maximum(z, 0.0) - az * rr
                    return carry2

                lax.fori_loop(0, _CHUNK, rows, 0)
                pltpu.sync_copy(hsv, acc.at[lidv.at[0]], add=True)
                return carry

            lax.fori_loop(0, nj, chunk_body, 0)
            plsc.subcore_barrier()

            @pl.when(s < _NS - 1)
            def _():
                pltpu.sync_copy(
                    acc.at[pl.ds(s * _STRIPE, _STRIPE)],
                    agg_hbm.at[pl.ds(bbase + s * _STRIPE, _STRIPE)])

            @pl.when(s == _NS - 1)
            def _():
                last = _B - (_NS - 1) * _STRIPE
                pltpu.sync_copy(
                    acc.at[pl.ds((_NS - 1) * _STRIPE, last)],
                    agg_hbm.at[pl.ds(bbase + (_NS - 1) * _STRIPE, last)])

            plsc.subcore_barrier()

    return edge_pass


def kernel(x, edge_index, edge_attr, Wn, bn, We, be, Wm, bm, Wu, bu,
           Wp, bp, Wv1, bv1, Wv2, bv2):
    N, ND = x.shape
    E = edge_attr.shape[0]
    L = Wm.shape[0]
    A = Wp.shape[1]
    assert Wm.shape[1] == 2 * _H + _EH and Wn.shape[1] == _H

    src = edge_index[0]
    dst = edge_index[1]

    rn = 2000
    assert N % rn == 0
    nb_n = N // rn
    re = 1600
    assert E % re == 0
    nb_e = E // re

    par = pltpu.CompilerParams(dimension_semantics=("parallel",))
    arb = pltpu.CompilerParams(dimension_semantics=("arbitrary",))

    h = pl.pallas_call(
        _enc_body,
        grid=(nb_n,),
        in_specs=[pl.BlockSpec((rn, ND), lambda i: (i, 0)),
                  _full_spec((ND, _H)), _full_spec((1, _H))],
        out_specs=pl.BlockSpec((rn, _H), lambda i: (i, 0)),
        out_shape=jax.ShapeDtypeStruct((N, _H), F32),
        compiler_params=par,
    )(x, Wn, bn.reshape(1, _H))

    ce_all = pl.pallas_call(
        _ce_body,
        grid=(nb_e,),
        in_specs=[pl.BlockSpec((re, 4), lambda i: (i, 0)),
                  _full_spec((4, _EH)), _full_spec((1, _EH)),
                  _full_spec((L, _EH, _H)), _full_spec((L, _H))],
        out_specs=pl.BlockSpec((L, re, _H), lambda i: (0, i, 0)),
        out_shape=jax.ShapeDtypeStruct((L, E, _H), F32),
        compiler_params=par,
    )(edge_attr, We, be.reshape(1, _EH), Wm[:, 2 * _H:, :], bm)

    prep = pl.pallas_call(
        _prep_body,
        grid=(nb_n,),
        in_specs=[pl.BlockSpec((rn, _H), lambda i: (i, 0)),
                  _full_spec((_H, _H)), _full_spec((_H, _H))],
        out_specs=[pl.BlockSpec((rn, _H), lambda i: (i, 0)),
                   pl.BlockSpec((rn, _H), lambda i: (i, 0))],
        out_shape=[jax.ShapeDtypeStruct((N, _H), F32),
                   jax.ShapeDtypeStruct((N, _H), F32)],
        compiler_params=par,
    )

    edge_pass = _make_edge_pass(N, E)

    upd = pl.pallas_call(
        _upd_body,
        grid=(nb_n,),
        in_specs=[pl.BlockSpec((rn, _H), lambda i: (i, 0)),
                  pl.BlockSpec((rn, _H), lambda i: (i, 0)),
                  _full_spec((2 * _H, _H)), _full_spec((1, _H))],
        out_specs=pl.BlockSpec((rn, _H), lambda i: (i, 0)),
        out_shape=jax.ShapeDtypeStruct((N, _H), F32),
        compiler_params=par,
    )

    for l in range(L):
        hs, hd = prep(h, Wm[l, :_H], Wm[l, _H:2 * _H])
        agg = edge_pass(src, dst, hs, hd, ce_all[l])
        h = upd(h, agg, Wu[l], bu[l].reshape(1, _H))

    logits, value = pl.pallas_call(
        functools.partial(_readout_body, nblocks=nb_n, n_rows=N),
        grid=(nb_n,),
        in_specs=[pl.BlockSpec((rn, _H), lambda i: (i, 0)),
                  _full_spec((2 * _H, A)), _full_spec((1, A)),
                  _full_spec((2 * _H, _H)), _full_spec((1, _H)),
                  _full_spec((_H, 1)), _full_spec((1, 1))],
        out_specs=[pl.BlockSpec((1, A), lambda i: (0, 0)),
                   pl.BlockSpec((1, 1), lambda i: (0, 0))],
        out_shape=[jax.ShapeDtypeStruct((1, A), F32),
                   jax.ShapeDtypeStruct((1, 1), F32)],
        scratch_shapes=[pltpu.VMEM((1, _H), F32), pltpu.VMEM((1, _H), F32)],
        compiler_params=arb,
    )(h, Wp, bp.reshape(1, A), Wv1, bv1.reshape(1, _H), Wv2,
      bv2.reshape(1, 1))

    return (logits, value)


# speedup vs baseline: 1.0164x; 1.0164x over previous
import functools

import jax
import jax.numpy as jnp
from jax import lax
from jax.experimental import pallas as pl
from jax.experimental.pallas import tpu as pltpu
from jax.experimental.pallas import tpu_sc as plsc

F32 = jnp.float32

_H = 128
_EH = 64
_NS = 16
_NC = 2
_NBKT = 4
_CHUNK = 64
_B = 12512
_ACC = 12544
_STRIPE = _ACC // _NS
_ZR = 16

_K1 = 0.7978845608028654
_K2 = _K1 * 0.044715
_C1 = -2.0 * _K1
_C2 = -2.0 * _K2


def _gelu_tc(v):
    return jax.nn.gelu(v)


def _enc_body(x_ref, wn_ref, bn_ref, o_ref):
    o_ref[...] = _gelu_tc(
        jnp.dot(x_ref[...], wn_ref[...], preferred_element_type=F32)
        + bn_ref[...])


def _ce_body(ea_ref, we_ref, be_ref, wme_ref, bm_ref, o_ref):
    L = wme_ref.shape[0]
    e = _gelu_tc(
        jnp.dot(ea_ref[...], we_ref[...], preferred_element_type=F32)
        + be_ref[...])
    for l in range(L):
        o_ref[l] = jnp.dot(e, wme_ref[l], preferred_element_type=F32) + bm_ref[l]


def _prep_body(h_ref, ws_ref, wd_ref, hs_ref, hd_ref):
    h = h_ref[...]
    hs_ref[...] = jnp.dot(h, ws_ref[...], preferred_element_type=F32)
    hd_ref[...] = jnp.dot(h, wd_ref[...], preferred_element_type=F32)


def _upd_body(h_ref, agg_ref, wu_ref, bu_ref, o_ref):
    h = h_ref[...]
    w = wu_ref[...]
    acc = (jnp.dot(h, w[:_H], preferred_element_type=F32)
           + jnp.dot(agg_ref[...], w[_H:], preferred_element_type=F32)
           + bu_ref[...])
    o_ref[...] = _gelu_tc(acc) + h


def _readout_body(h_ref, wp_ref, bp_ref, wv1_ref, bv1_ref, wv2_ref, bv2_ref,
                  lo_ref, vo_ref, sum_ref, max_ref, *, nblocks, n_rows):
    i = pl.program_id(0)
    blk = h_ref[...]
    bsum = jnp.sum(blk, axis=0, keepdims=True)
    bmax = jnp.max(blk, axis=0, keepdims=True)

    @pl.when(i == 0)
    def _():
        sum_ref[...] = bsum
        max_ref[...] = bmax

    @pl.when(i > 0)
    def _():
        sum_ref[...] = sum_ref[...] + bsum
        max_ref[...] = jnp.maximum(max_ref[...], bmax)

    @pl.when(i == nblocks - 1)
    def _():
        g = jnp.concatenate([sum_ref[...] / float(n_rows), max_ref[...]],
                            axis=1)
        lo_ref[...] = (jnp.dot(g, wp_ref[...], preferred_element_type=F32)
                       + bp_ref[...])
        v = _gelu_tc(jnp.dot(g, wv1_ref[...], preferred_element_type=F32)
                     + bv1_ref[...])
        vo_ref[...] = (jnp.dot(v, wv2_ref[...], preferred_element_type=F32)
                       + bv2_ref[...])


def _full_spec(shape):
    return pl.BlockSpec(shape, lambda i: tuple(0 for _ in shape))


def _make_edge_pass(N, E):
    assert E % _CHUNK == 0 and N <= _NBKT * _B
    nch = E // _CHUNK
    base_nj, rem = divmod(nch, _NS)
    mesh = plsc.VectorSubcoreMesh(core_axis_name="c", subcore_axis_name="s")

    @functools.partial(
        pl.kernel,
        out_type=jax.ShapeDtypeStruct((_NBKT * _B, _H), F32),
        mesh=mesh,
        scratch_types=[
            pltpu.VMEM((_CHUNK,), jnp.int32),
            pltpu.VMEM((1, _CHUNK), jnp.int32),
            pltpu.VMEM((1, _CHUNK), jnp.int32),
            pltpu.VMEM((_CHUNK, _H), F32),
            pltpu.VMEM((_CHUNK, _H), F32),
            pltpu.VMEM((_CHUNK, _H), F32),
            pltpu.VMEM((_ZR, _H), F32),
            pltpu.VMEM_SHARED((_ACC, _H), F32),
            pltpu.SemaphoreType.DMA,
        ],
    )
    def edge_pass(src_hbm, dst_hbm, hs_hbm, hd_hbm, ce_hbm, agg_hbm,
                  srcv, dstv, lidv, cev, hsv, hdv, zbuf, acc, sem):
        c = lax.axis_index("c")
        s = lax.axis_index("s")
        nj = base_nj + jnp.where(s < rem, 1, 0)

        zv = jnp.zeros((16,), F32)

        def zb(r, carry):
            for cb in range(_H // 16):
                zbuf[r, pl.ds(cb * 16, 16)] = zv
            return carry

        lax.fori_loop(0, _ZR, zb, 0)
        nzc = _STRIPE // _ZR

        for p in range(_NBKT // _NC):
            bkt = p * _NC + c
            bbase = bkt * _B

            def zc(t, carry):
                pltpu.sync_copy(zbuf, acc.at[pl.ds(s * _STRIPE + t * _ZR,
                                                   _ZR)])
                return carry

            lax.fori_loop(0, nzc, zc, 0)
            plsc.subcore_barrier()

            def chunk_body(j, carry):
                base = (s + _NS * j) * _CHUNK
                cp_s = pltpu.async_copy(src_hbm.at[pl.ds(base, _CHUNK)],
                                        srcv, sem)
                cp_d = pltpu.async_copy(dst_hbm.at[pl.ds(base, _CHUNK)],
                                        dstv.at[0], sem)
                cp_c = pltpu.async_copy(ce_hbm.at[pl.ds(base, _CHUNK)],
                                        cev, sem)
                cp_s.wait()
                cp_d.wait()

                def lids(t, carry2):
                    sl = pl.ds(t * 16, 16)
                    lid = dstv[0, sl] - bbase
                    ok = (lid >= 0) & (lid < _B)
                    lidv[0, sl] = jnp.where(ok, lid, _B)
                    return carry2

                lax.fori_loop(0, _CHUNK // 16, lids, 0)
                cp_hs = pltpu.async_copy(hs_hbm.at[srcv], hsv, sem)
                cp_hd = pltpu.async_copy(hd_hbm.at[dstv.at[0]], hdv, sem)
                cp_c.wait()
                cp_hs.wait()
                cp_hd.wait()

                def rows(r, carry2):
                    for cb in range(_H // 16):
                        sl = pl.ds(cb * 16, 16)
                        z = hsv[r, sl] + hdv[r, sl] + cev[r, sl]
                        t2 = _C1 + _C2 * (z * z)
                        eu = jnp.exp(z * t2)
                        hsv[r, sl] = z / (1.0 + eu)
                    return carry2

                lax.fori_loop(0, _CHUNK, rows, 0)
                pltpu.sync_copy(hsv, acc.at[lidv.at[0]], add=True)
                return carry

            lax.fori_loop(0, nj, chunk_body, 0)
            plsc.subcore_barrier()

            @pl.when(s < _NS - 1)
            def _():
                pltpu.sync_copy(
                    acc.at[pl.ds(s * _STRIPE, _STRIPE)],
                    agg_hbm.at[pl.ds(bbase + s * _STRIPE, _STRIPE)])

            @pl.when(s == _NS - 1)
            def _():
                last = _B - (_NS - 1) * _STRIPE
                pltpu.sync_copy(
                    acc.at[pl.ds((_NS - 1) * _STRIPE, last)],
                    agg_hbm.at[pl.ds(bbase + (_NS - 1) * _STRIPE, last)])

            plsc.subcore_barrier()

    return edge_pass


def kernel(x, edge_index, edge_attr, Wn, bn, We, be, Wm, bm, Wu, bu,
           Wp, bp, Wv1, bv1, Wv2, bv2):
    N, ND = x.shape
    E = edge_attr.shape[0]
    L = Wm.shape[0]
    A = Wp.shape[1]
    assert Wm.shape[1] == 2 * _H + _EH and Wn.shape[1] == _H

    src = edge_index[0]
    dst = edge_index[1]

    rn = 2000
    assert N % rn == 0
    nb_n = N // rn
    re = 1600
    assert E % re == 0
    nb_e = E // re

    par = pltpu.CompilerParams(dimension_semantics=("parallel",))
    arb = pltpu.CompilerParams(dimension_semantics=("arbitrary",))

    h = pl.pallas_call(
        _enc_body,
        grid=(nb_n,),
        in_specs=[pl.BlockSpec((rn, ND), lambda i: (i, 0)),
                  _full_spec((ND, _H)), _full_spec((1, _H))],
        out_specs=pl.BlockSpec((rn, _H), lambda i: (i, 0)),
        out_shape=jax.ShapeDtypeStruct((N, _H), F32),
        compiler_params=par,
    )(x, Wn, bn.reshape(1, _H))

    ce_all = pl.pallas_call(
        _ce_body,
        grid=(nb_e,),
        in_specs=[pl.BlockSpec((re, 4), lambda i: (i, 0)),
                  _full_spec((4, _EH)), _full_spec((1, _EH)),
                  _full_spec((L, _EH, _H)), _full_spec((L, _H))],
        out_specs=pl.BlockSpec((L, re, _H), lambda i: (0, i, 0)),
        out_shape=jax.ShapeDtypeStruct((L, E, _H), F32),
        compiler_params=par,
    )(edge_attr, We, be.reshape(1, _EH), Wm[:, 2 * _H:, :], bm)

    prep = pl.pallas_call(
        _prep_body,
        grid=(nb_n,),
        in_specs=[pl.BlockSpec((rn, _H), lambda i: (i, 0)),
                  _full_spec((_H, _H)), _full_spec((_H, _H))],
        out_specs=[pl.BlockSpec((rn, _H), lambda i: (i, 0)),
                   pl.BlockSpec((rn, _H), lambda i: (i, 0))],
        out_shape=[jax.ShapeDtypeStruct((N, _H), F32),
                   jax.ShapeDtypeStruct((N, _H), F32)],
        compiler_params=par,
    )

    edge_pass = _make_edge_pass(N, E)

    upd = pl.pallas_call(
        _upd_body,
        grid=(nb_n,),
        in_specs=[pl.BlockSpec((rn, _H), lambda i: (i, 0)),
                  pl.BlockSpec((rn, _H), lambda i: (i, 0)),
                  _full_spec((2 * _H, _H)), _full_spec((1, _H))],
        out_specs=pl.BlockSpec((rn, _H), lambda i: (i, 0)),
        out_shape=jax.ShapeDtypeStruct((N, _H), F32),
        compiler_params=par,
    )

    for l in range(L):
        hs, hd = prep(h, Wm[l, :_H], Wm[l, _H:2 * _H])
        agg = edge_pass(src, dst, hs, hd, ce_all[l])
        h = upd(h, agg, Wu[l], bu[l].reshape(1, _H))

    logits, value = pl.pallas_call(
        functools.partial(_readout_body, nblocks=nb_n, n_rows=N),
        grid=(nb_n,),
        in_specs=[pl.BlockSpec((rn, _H), lambda i: (i, 0)),
                  _full_spec((2 * _H, A)), _full_spec((1, A)),
                  _full_spec((2 * _H, _H)), _full_spec((1, _H)),
                  _full_spec((_H, 1)), _full_spec((1, 1))],
        out_specs=[pl.BlockSpec((1, A), lambda i: (0, 0)),
                   pl.BlockSpec((1, 1), lambda i: (0, 0))],
        out_shape=[jax.ShapeDtypeStruct((1, A), F32),
                   jax.ShapeDtypeStruct((1, 1), F32)],
        scratch_shapes=[pltpu.VMEM((1, _H), F32), pltpu.VMEM((1, _H), F32)],
        compiler_params=arb,
    )(h, Wp, bp.reshape(1, A), Wv1, bv1.reshape(1, _H), Wv2,
      bv2.reshape(1, 1))

    return (logits, value)
